# baseline (speedup 1.0000x reference)
import functools

import jax
import jax.numpy as jnp
from jax import lax
from jax.experimental import pallas as pl
from jax.experimental.pallas import tpu as pltpu
from jax.experimental.pallas import tpu_sc as plsc

NC = 2
NS = 16
NW = NC * NS
CH = 64
NBUF = 2
W = 256


@functools.lru_cache(maxsize=None)
def _build(V, D, B):
    assert B % (NW * CH * NBUF) == 0 and D % 16 == 0 and V >= W
    b_per_w = B // NW
    n_ch = b_per_w // CH
    n_super = n_ch // NBUF
    mesh = plsc.VectorSubcoreMesh(core_axis_name="c", subcore_axis_name="s")

    @functools.partial(
        pl.kernel,
        out_type=jax.ShapeDtypeStruct((B * D,), jnp.float32),
        mesh=mesh,
        scratch_types=[
            pltpu.VMEM((n_ch, CH), jnp.int32),
            pltpu.VMEM((W * D,), jnp.float32),
            [pltpu.VMEM((CH * D,), jnp.float32) for _ in range(NBUF)],
            [pltpu.SemaphoreType.DMA for _ in range(NBUF)],
        ],
    )
    def gather_kernel(table_hbm, idx_hbm, out_hbm, idx_v, win, rows, osem):
        wid = lax.axis_index("s") * NC + lax.axis_index("c")
        pltpu.sync_copy(idx_hbm.at[wid], idx_v)
        base = wid * b_per_w
        wmin = idx_v[0, pl.ds(0, 16)][0]
        wmax = idx_v[n_ch - 1, pl.ds(CH - 16, 16)][15]
        wstart = jnp.maximum(jnp.minimum(wmin, V - W), 0)

        def m8(x):
            return pl.multiple_of(x, 8)

        def out_slice(g):
            return out_hbm.at[pl.ds(m8((base + g * CH) * D), CH * D)]

        @pl.when(wmax - wstart < W)
        def _fast():
            pltpu.sync_copy(
                table_hbm.at[pl.ds(m8(wstart * D), W * D)], win)

            def super_body(s, carry):
                for b in range(NBUF):
                    g = s * NBUF + b

                    @pl.when(s > 0)
                    def _():
                        pltpu.make_async_copy(
                            rows[b], out_hbm.at[pl.ds(0, CH * D)],
                            osem[b]).wait()

                    def group_body(grp, c2, b=b, g=g):
                        pvec = (idx_v[g, pl.ds(grp * 16, 16)] - wstart) * D
                        rbase = grp * (16 * D)
                        for l in range(16):
                            pb = pvec[l]
                            for c in range(D // 16):
                                rows[b][pl.ds(
                                    rbase + l * D + c * 16, 16)] = (
                                    win[pl.ds(pb + c * 16, 16)])
                        return c2

                    lax.fori_loop(0, CH // 16, group_body, 0)
                    pltpu.async_copy(rows[b], out_slice(g), osem[b])
                return carry

            lax.fori_loop(0, n_super, super_body, 0)
            for b in range(NBUF):
                pltpu.make_async_copy(
                    rows[b], out_hbm.at[pl.ds(0, CH * D)], osem[b]).wait()

        @pl.when(wmax - wstart >= W)
        def _general():
            def fb_group(grp, carry):
                g = grp // (CH // 16)
                a0 = (grp % (CH // 16)) * 16
                pvec = idx_v[g, pl.ds(a0, 16)] * D
                for l in range(16):
                    pltpu.async_copy(
                        table_hbm.at[pl.ds(m8(pvec[l]), D)],
                        rows[0].at[pl.ds(l * D, D)], osem[0])
                for l in range(16):
                    pltpu.make_async_copy(
                        table_hbm.at[pl.ds(0, D)],
                        rows[0].at[pl.ds(0, D)], osem[0]).wait()
                pltpu.sync_copy(
                    rows[0].at[pl.ds(0, 16 * D)],
                    out_hbm.at[pl.ds(m8((base + g * CH + a0) * D),
                                     16 * D)])
                return carry

            lax.fori_loop(0, b_per_w // 16, fb_group, 0)

    return gather_kernel


def kernel(system_features, batch_index):
    V, D = system_features.shape
    (B,) = batch_index.shape
    idx = batch_index.astype(jnp.int32).reshape(NW, B // (NW * CH), CH)
    out = _build(V, D, B)(system_features.reshape(-1), idx)
    return out.reshape(B, D)


# speedup vs baseline: 1.7755x; 1.7755x over previous
import functools

import jax
import jax.numpy as jnp
from jax import lax
from jax.experimental import pallas as pl
from jax.experimental.pallas import tpu as pltpu
from jax.experimental.pallas import tpu_sc as plsc

NC = 2
NS = 16
NW = NC * NS
W = 256
LAGG = 4


@functools.lru_cache(maxsize=None)
def _build(V, D, B):
    assert B % (NW * 16) == 0 and D % 16 == 0 and V >= W
    b_per_w = B // NW
    n_grp = b_per_w // 16
    mesh = plsc.VectorSubcoreMesh(core_axis_name="c", subcore_axis_name="s")

    @functools.partial(
        pl.kernel,
        out_type=jax.ShapeDtypeStruct((B * D,), jnp.float32),
        mesh=mesh,
        scratch_types=[
            pltpu.VMEM((n_grp, 16), jnp.int32),
            pltpu.VMEM((W * D,), jnp.float32),
            pltpu.VMEM((16 * D,), jnp.float32),
            pltpu.SemaphoreType.DMA,
        ],
    )
    def gather_kernel(table_hbm, idx_hbm, out_hbm, idx_v, win, stage, sem):
        wid = lax.axis_index("s") * NC + lax.axis_index("c")
        pltpu.sync_copy(idx_hbm.at[wid], idx_v)
        base = wid * b_per_w
        wmin = idx_v[0, pl.ds(0, 16)][0]
        wmax = idx_v[n_grp - 1, pl.ds(0, 16)][15]
        wstart = jnp.maximum(jnp.minimum(wmin, V - W), 0)

        def m8(x):
            return pl.multiple_of(x, 8)

        def drain16():
            for _ in range(16):
                pltpu.make_async_copy(
                    win.at[pl.ds(0, D)], out_hbm.at[pl.ds(0, D)],
                    sem).wait()

        @pl.when(wmax - wstart < W)
        def _fast():
            pltpu.sync_copy(
                table_hbm.at[pl.ds(m8(wstart * D), W * D)], win)

            def grp_body(grp, carry):
                pvec = (idx_v[grp, pl.ds(0, 16)] - wstart) * D
                abase = (base + grp * 16) * D
                for l in range(16):
                    pltpu.async_copy(
                        win.at[pl.ds(m8(pvec[l]), D)],
                        out_hbm.at[pl.ds(m8(abase + l * D), D)], sem)

                @pl.when(grp >= LAGG)
                def _():
                    drain16()

                return carry

            lax.fori_loop(0, n_grp, grp_body, 0)

            def tail_body(i, carry):
                drain16()
                return carry

            lax.fori_loop(0, min(LAGG, n_grp), tail_body, 0)

        @pl.when(wmax - wstart >= W)
        def _general():
            def fb_group(grp, carry):
                pvec = idx_v[grp, pl.ds(0, 16)] * D
                for l in range(16):
                    pltpu.async_copy(
                        table_hbm.at[pl.ds(m8(pvec[l]), D)],
                        stage.at[pl.ds(l * D, D)], sem)
                for l in range(16):
                    pltpu.make_async_copy(
                        table_hbm.at[pl.ds(0, D)],
                        stage.at[pl.ds(0, D)], sem).wait()
                pltpu.sync_copy(
                    stage,
                    out_hbm.at[pl.ds(m8((base + grp * 16) * D), 16 * D)])
                return carry

            lax.fori_loop(0, n_grp, fb_group, 0)

    return gather_kernel


def kernel(system_features, batch_index):
    V, D = system_features.shape
    (B,) = batch_index.shape
    idx = batch_index.astype(jnp.int32).reshape(NW, B // (NW * 16), 16)
    out = _build(V, D, B)(system_features.reshape(-1), idx)
    return out.reshape(B, D)
